# Initial kernel scaffold; baseline (speedup 1.0000x reference)
import jax
import jax.numpy as jnp
from jax.experimental import pallas as pl


def kernel(joints):
    raise NotImplementedError("write your pallas kernel here")


# speedup vs baseline: 11.1455x; 11.1455x over previous
import functools

import jax
import jax.numpy as jnp
import numpy as np
from jax.experimental import pallas as pl
from jax.experimental.pallas import tpu as pltpu


def _pairs():
    j1, j2 = [], []
    for a in range(11):
        for b in range(11, 41):
            j1.append(a)
            j2.append(b)
    for a in range(11, 41):
        for b in range(a + 1, 41):
            if (a - 11) // 6 != (b - 11) // 6:
                j1.append(a)
                j2.append(b)
    return np.asarray(j1, dtype=np.int32), np.asarray(j2, dtype=np.int32)


_J1, _J2 = _pairs()
_NPAIR = _J1.shape[0]
_PPAD = 704
_MIN_SQ = 36.0


def _diff_matrix():
    d = np.zeros((123, 3 * _PPAD), dtype=np.float32)
    for p in range(_NPAIR):
        for c in range(3):
            d[3 * _J1[p] + c, c * _PPAD + p] += 1.0
            d[3 * _J2[p] + c, c * _PPAD + p] -= 1.0
    return d


_D = _diff_matrix()


def _body(x_ref, d_ref, o_ref):
    i = pl.program_id(0)

    @pl.when(i == 0)
    def _():
        o_ref[0, 0] = 0.0

    diff = jnp.dot(x_ref[:], d_ref[:], preferred_element_type=jnp.float32)
    dx = diff[:, :_PPAD]
    dy = diff[:, _PPAD:2 * _PPAD]
    dz = diff[:, 2 * _PPAD:]
    sq = dx * dx + dy * dy + dz * dz
    lane = jax.lax.broadcasted_iota(jnp.int32, sq.shape, 1)
    loss = jnp.where(lane < _NPAIR, jnp.maximum(_MIN_SQ - sq, 0.0), 0.0)
    o_ref[0, 0] += jnp.sum(loss)


@jax.jit
def kernel(joints):
    num_batch = joints.shape[0]
    bb = 512
    grid = num_batch // bb
    d = jnp.asarray(_D)
    out = pl.pallas_call(
        _body,
        grid=(grid,),
        in_specs=[
            pl.BlockSpec((bb, 123), lambda i: (i, 0)),
            pl.BlockSpec((123, 3 * _PPAD), lambda i: (0, 0)),
        ],
        out_specs=pl.BlockSpec(memory_space=pltpu.SMEM),
        out_shape=jax.ShapeDtypeStruct((1, 1), jnp.float32),
    )(joints, d)
    return out[0, 0]
